# baseline (speedup 1.0000x reference)
import functools

import jax
import jax.numpy as jnp
from jax import lax
from jax.experimental import pallas as pl
from jax.experimental.pallas import tpu as pltpu
from jax.experimental.pallas import tpu_sc as plsc

EMBED = 64
NUM_CORES = 2
NUM_SUBCORES = 16
NUM_WORKERS = NUM_CORES * NUM_SUBCORES
CHUNK = 640


def _sc_gather(flat_idx, table, num_indices):
    b_per_w = num_indices // NUM_WORKERS
    n_chunks = b_per_w // CHUNK
    mesh = plsc.VectorSubcoreMesh(core_axis_name="c", subcore_axis_name="s")

    @functools.partial(
        pl.kernel,
        out_type=jax.ShapeDtypeStruct((num_indices, EMBED), table.dtype),
        mesh=mesh,
        scratch_types=[
            pltpu.VMEM((CHUNK,), jnp.int32),
            pltpu.VMEM((CHUNK, EMBED), jnp.float32),
            pltpu.SemaphoreType.DMA,
        ],
        compiler_params=pltpu.CompilerParams(use_tc_tiling_on_sc=False),
    )
    def kfn(table_hbm, idx_hbm, out_hbm, idx_v, rows_v, sem):
        wid = lax.axis_index("s") * NUM_CORES + lax.axis_index("c")
        base = wid * b_per_w

        @pl.loop(0, n_chunks)
        def _(j):
            off = base + j * CHUNK
            pltpu.sync_copy(idx_hbm.at[pl.ds(off, CHUNK)], idx_v)
            pltpu.async_copy(table_hbm.at[idx_v], rows_v, sem).wait()
            pltpu.sync_copy(rows_v, out_hbm.at[pl.ds(off, CHUNK)])

    return kfn(table, flat_idx)


def kernel(x, table):
    batch, seq = x.shape
    num_indices = batch * seq
    flat_idx = x.reshape(num_indices).astype(jnp.int32)
    out = _sc_gather(flat_idx, table, num_indices)
    return out.reshape(batch, seq, EMBED)


# speedup vs baseline: 1.0105x; 1.0105x over previous
import functools

import jax
import jax.numpy as jnp
from jax import lax
from jax.experimental import pallas as pl
from jax.experimental.pallas import tpu as pltpu
from jax.experimental.pallas import tpu_sc as plsc

EMBED = 64
NUM_CORES = 2
NUM_SUBCORES = 16
NUM_WORKERS = NUM_CORES * NUM_SUBCORES
CHUNK = 640


def _sc_gather(flat_idx, table, num_indices):
    b_per_w = num_indices // NUM_WORKERS
    n_chunks = b_per_w // CHUNK
    mesh = plsc.VectorSubcoreMesh(core_axis_name="c", subcore_axis_name="s")

    @functools.partial(
        pl.kernel,
        out_type=jax.ShapeDtypeStruct((num_indices, EMBED), table.dtype),
        mesh=mesh,
        scratch_types=[
            pltpu.VMEM((b_per_w,), jnp.int32),
            pltpu.VMEM((CHUNK, EMBED), jnp.float32),
            pltpu.VMEM((CHUNK, EMBED), jnp.float32),
            pltpu.SemaphoreType.DMA,
            pltpu.SemaphoreType.DMA,
            pltpu.SemaphoreType.DMA,
            pltpu.SemaphoreType.DMA,
        ],
        compiler_params=pltpu.CompilerParams(use_tc_tiling_on_sc=False),
    )
    def kfn(table_hbm, idx_hbm, out_hbm, idx_all, rows0, rows1,
            sg0, sg1, so0, so1):
        wid = lax.axis_index("s") * NUM_CORES + lax.axis_index("c")
        base = wid * b_per_w
        rows = (rows0, rows1)
        sg = (sg0, sg1)
        so = (so0, so1)

        pltpu.sync_copy(idx_hbm.at[pl.ds(base, b_per_w)], idx_all)

        def gather(j, b):
            return pltpu.make_async_copy(
                table_hbm.at[idx_all.at[pl.ds(j * CHUNK, CHUNK)]],
                rows[b], sg[b])

        def writeback(j, b):
            return pltpu.make_async_copy(
                rows[b], out_hbm.at[pl.ds(base + j * CHUNK, CHUNK)], so[b])

        gather(0, 0).start()
        for j in range(1, n_chunks):
            b = j % 2
            if j >= 2:
                writeback(j - 2, b).wait()
            gather(j, b).start()
            gather(j - 1, 1 - b).wait()
            writeback(j - 1, 1 - b).start()
        last = n_chunks - 1
        gather(last, last % 2).wait()
        writeback(last, last % 2).start()
        writeback(last - 1, (last - 1) % 2).wait()
        writeback(last, last % 2).wait()

    return kfn(table, flat_idx)


def kernel(x, table):
    batch, seq = x.shape
    num_indices = batch * seq
    flat_idx = x.reshape(num_indices).astype(jnp.int32)
    out = _sc_gather(flat_idx, table, num_indices)
    return out.reshape(batch, seq, EMBED)
